# baseline (speedup 1.0000x reference)
import functools

import jax
import jax.numpy as jnp
from jax import lax
from jax.experimental import pallas as pl
from jax.experimental.pallas import tpu as pltpu

_R = 200
_CACHE_CHUNKS = 4


def _fc1_body(x_ref, w1_ref, b1_ref, a_ref):
    a_ref[...] = (
        jnp.dot(x_ref[...], w1_ref[...], preferred_element_type=jnp.float32)
        + b1_ref[...]
    )


def _main_body(a_ref, adj_ref, w2_ref, b2_ref, watt_ref, batt_ref, wcls_ref,
               bcls_ref, out_ref,
               ring0, ring1, cache_ref, h1_ref, bm_ref,
               sem0, sem1, semc, *, n, h, c):
    r = _R
    nring = (n - _CACHE_CHUNKS * r) // r
    cbase = nring * r
    f32 = jnp.float32

    def start_rs(row0, ring, sem):
        pltpu.make_async_copy(adj_ref.at[pl.ds(row0, r), :], ring, sem).start()

    def wait_rs(ring, sem):
        pltpu.make_async_copy(adj_ref.at[pl.ds(0, r), :], ring, sem).wait()

    def p1_compute(row0, blk):
        h1 = jnp.maximum(
            jnp.dot(blk, a_ref[...], preferred_element_type=f32), 0.0
        )
        h1_ref[pl.ds(row0, r), :] = h1
        bm_ref[pl.ds(row0, r), :] = (
            jnp.dot(h1, w2_ref[...], preferred_element_type=f32) + b2_ref[...]
        )

    def p2_compute(row0, blk, carry):
        m, d, g = carry
        h2 = jnp.maximum(
            jnp.dot(blk, bm_ref[...], preferred_element_type=f32), 0.0
        )
        hrow = h1_ref[pl.ds(row0, r), :] + h2
        s = (
            jnp.dot(hrow, watt_ref[...], preferred_element_type=f32)
            + batt_ref[0, 0]
        )
        m2 = jnp.maximum(m, jnp.max(s))
        sc = jnp.exp(m - m2)
        e = jnp.exp(s - m2)
        return (m2, d * sc + jnp.sum(e),
                g * sc + jnp.sum(e * hrow, axis=0, keepdims=True))

    start_rs(0, ring0, sem0)
    start_rs(r, ring1, sem1)

    def body1(i, _):
        row0 = 2 * i * r
        wait_rs(ring0, sem0)
        p1_compute(row0, ring0[...])
        start_rs(row0 + 2 * r, ring0, sem0)
        wait_rs(ring1, sem1)
        p1_compute(row0 + r, ring1[...])
        start_rs(row0 + 3 * r, ring1, sem1)
        return 0

    lax.fori_loop(0, (nring - 4) // 2, body1, 0)

    wait_rs(ring0, sem0)
    p1_compute((nring - 4) * r, ring0[...])
    start_rs((nring - 2) * r, ring0, sem0)
    wait_rs(ring1, sem1)
    p1_compute((nring - 3) * r, ring1[...])
    start_rs((nring - 1) * r, ring1, sem1)
    wait_rs(ring0, sem0)
    p1_compute((nring - 2) * r, ring0[...])
    pltpu.make_async_copy(
        adj_ref.at[pl.ds(cbase, r), :], cache_ref.at[pl.ds(0, r), :], semc
    ).start()
    pltpu.make_async_copy(
        adj_ref.at[pl.ds(cbase + r, r), :], cache_ref.at[pl.ds(r, r), :], semc
    ).start()
    wait_rs(ring1, sem1)
    p1_compute((nring - 1) * r, ring1[...])
    pltpu.make_async_copy(
        adj_ref.at[pl.ds(cbase + 2 * r, r), :],
        cache_ref.at[pl.ds(2 * r, r), :], semc
    ).start()
    pltpu.make_async_copy(
        adj_ref.at[pl.ds(cbase + 3 * r, r), :],
        cache_ref.at[pl.ds(3 * r, r), :], semc
    ).start()
    start_rs(0, ring0, sem0)
    start_rs(r, ring1, sem1)

    for i in range(_CACHE_CHUNKS):
        pltpu.make_async_copy(
            adj_ref.at[pl.ds(cbase + i * r, r), :],
            cache_ref.at[pl.ds(i * r, r), :], semc
        ).wait()
        p1_compute(cbase + i * r, cache_ref[pl.ds(i * r, r), :])

    carry = (jnp.float32(-jnp.inf), jnp.float32(0.0),
             jnp.zeros((1, h), f32))
    for i in range(_CACHE_CHUNKS):
        carry = p2_compute(cbase + i * r, cache_ref[pl.ds(i * r, r), :], carry)

    def body2(i, carry):
        row0 = 2 * i * r
        wait_rs(ring0, sem0)
        carry = p2_compute(row0, ring0[...], carry)
        start_rs(row0 + 2 * r, ring0, sem0)
        wait_rs(ring1, sem1)
        carry = p2_compute(row0 + r, ring1[...], carry)
        start_rs(row0 + 3 * r, ring1, sem1)
        return carry

    carry = lax.fori_loop(0, (nring - 2) // 2, body2, carry)

    wait_rs(ring0, sem0)
    carry = p2_compute((nring - 2) * r, ring0[...], carry)
    wait_rs(ring1, sem1)
    carry = p2_compute((nring - 1) * r, ring1[...], carry)

    m, d, g = carry
    out_ref[...] = (
        jnp.dot(g / d, wcls_ref[...], preferred_element_type=f32)
        + bcls_ref[...]
    )


def kernel(x, adj, W1, b1, W2, b2, Watt, batt, Wcls, bcls):
    N, DIN = x.shape
    H = W1.shape[1]
    C = Wcls.shape[1]
    f32 = jnp.float32

    a = pl.pallas_call(
        _fc1_body,
        out_shape=jax.ShapeDtypeStruct((N, H), f32),
    )(x, W1, b1.reshape(1, H))

    out = pl.pallas_call(
        functools.partial(_main_body, n=N, h=H, c=C),
        in_specs=[
            pl.BlockSpec(memory_space=pltpu.MemorySpace.VMEM),
            pl.BlockSpec(memory_space=pl.ANY),
            pl.BlockSpec(memory_space=pltpu.MemorySpace.VMEM),
            pl.BlockSpec(memory_space=pltpu.MemorySpace.VMEM),
            pl.BlockSpec(memory_space=pltpu.MemorySpace.VMEM),
            pl.BlockSpec(memory_space=pltpu.MemorySpace.VMEM),
            pl.BlockSpec(memory_space=pltpu.MemorySpace.VMEM),
            pl.BlockSpec(memory_space=pltpu.MemorySpace.VMEM),
        ],
        out_shape=jax.ShapeDtypeStruct((1, C), f32),
        scratch_shapes=[
            pltpu.VMEM((_R, N), f32),
            pltpu.VMEM((_R, N), f32),
            pltpu.VMEM((_CACHE_CHUNKS * _R, N), f32),
            pltpu.VMEM((N, H), f32),
            pltpu.VMEM((N, H), f32),
            pltpu.SemaphoreType.DMA,
            pltpu.SemaphoreType.DMA,
            pltpu.SemaphoreType.DMA,
        ],
        compiler_params=pltpu.CompilerParams(
            vmem_limit_bytes=64 * 1024 * 1024,
        ),
    )(a, adj, W2, b2.reshape(1, H), Watt, batt.reshape(1, 1), Wcls,
      bcls.reshape(1, C))

    return out.reshape(C)


# speedup vs baseline: 1.0851x; 1.0851x over previous
import functools

import jax
import jax.numpy as jnp
from jax import lax
from jax.experimental import pallas as pl
from jax.experimental.pallas import tpu as pltpu

_R = 400
_CACHE_CHUNKS = 1


def _fc1_body(x_ref, w1_ref, b1_ref, a_ref):
    a_ref[...] = (
        jnp.dot(x_ref[...], w1_ref[...], preferred_element_type=jnp.float32)
        + b1_ref[...]
    )


def _main_body(a_ref, adj_ref, w2_ref, b2_ref, watt_ref, batt_ref, wcls_ref,
               bcls_ref, out_ref,
               ring0, ring1, cache_ref, h1_ref, bm_ref,
               sem0, sem1, semc, *, n, h, c):
    r = _R
    nring = (n - _CACHE_CHUNKS * r) // r
    cbase = nring * r
    f32 = jnp.float32

    def start_rs(row0, ring, sem):
        pltpu.make_async_copy(adj_ref.at[pl.ds(row0, r), :], ring, sem).start()

    def wait_rs(ring, sem):
        pltpu.make_async_copy(adj_ref.at[pl.ds(0, r), :], ring, sem).wait()

    def p1_compute(row0, blk):
        h1 = jnp.maximum(
            jnp.dot(blk, a_ref[...], preferred_element_type=f32), 0.0
        )
        h1_ref[pl.ds(row0, r), :] = h1
        bm_ref[pl.ds(row0, r), :] = (
            jnp.dot(h1, w2_ref[...], preferred_element_type=f32) + b2_ref[...]
        )

    def p2_compute(row0, blk, carry):
        m, d, g = carry
        h2 = jnp.maximum(
            jnp.dot(blk, bm_ref[...], preferred_element_type=f32), 0.0
        )
        hrow = h1_ref[pl.ds(row0, r), :] + h2
        s = (
            jnp.dot(hrow, watt_ref[...], preferred_element_type=f32)
            + batt_ref[0, 0]
        )
        m2 = jnp.maximum(m, jnp.max(s))
        sc = jnp.exp(m - m2)
        e = jnp.exp(s - m2)
        return (m2, d * sc + jnp.sum(e),
                g * sc + jnp.sum(e * hrow, axis=0, keepdims=True))

    start_rs(0, ring0, sem0)
    start_rs(r, ring1, sem1)

    def body1(i, _):
        row0 = 2 * i * r
        wait_rs(ring0, sem0)
        p1_compute(row0, ring0[...])
        start_rs(row0 + 2 * r, ring0, sem0)
        wait_rs(ring1, sem1)
        p1_compute(row0 + r, ring1[...])
        start_rs(row0 + 3 * r, ring1, sem1)
        return 0

    lax.fori_loop(0, (nring - 4) // 2, body1, 0)

    wait_rs(ring0, sem0)
    p1_compute((nring - 4) * r, ring0[...])
    start_rs((nring - 2) * r, ring0, sem0)
    wait_rs(ring1, sem1)
    p1_compute((nring - 3) * r, ring1[...])
    start_rs((nring - 1) * r, ring1, sem1)
    wait_rs(ring0, sem0)
    p1_compute((nring - 2) * r, ring0[...])
    for i in range(_CACHE_CHUNKS):
        pltpu.make_async_copy(
            adj_ref.at[pl.ds(cbase + i * r, r), :],
            cache_ref.at[pl.ds(i * r, r), :], semc
        ).start()
    wait_rs(ring1, sem1)
    p1_compute((nring - 1) * r, ring1[...])
    start_rs(0, ring0, sem0)
    start_rs(r, ring1, sem1)

    for i in range(_CACHE_CHUNKS):
        pltpu.make_async_copy(
            adj_ref.at[pl.ds(cbase + i * r, r), :],
            cache_ref.at[pl.ds(i * r, r), :], semc
        ).wait()
        p1_compute(cbase + i * r, cache_ref[pl.ds(i * r, r), :])

    carry = (jnp.float32(-jnp.inf), jnp.float32(0.0),
             jnp.zeros((1, h), f32))
    for i in range(_CACHE_CHUNKS):
        carry = p2_compute(cbase + i * r, cache_ref[pl.ds(i * r, r), :], carry)

    def body2(i, carry):
        row0 = 2 * i * r
        wait_rs(ring0, sem0)
        carry = p2_compute(row0, ring0[...], carry)
        start_rs(row0 + 2 * r, ring0, sem0)
        wait_rs(ring1, sem1)
        carry = p2_compute(row0 + r, ring1[...], carry)
        start_rs(row0 + 3 * r, ring1, sem1)
        return carry

    carry = lax.fori_loop(0, (nring - 2) // 2, body2, carry)

    wait_rs(ring0, sem0)
    carry = p2_compute((nring - 2) * r, ring0[...], carry)
    wait_rs(ring1, sem1)
    carry = p2_compute((nring - 1) * r, ring1[...], carry)

    m, d, g = carry
    out_ref[...] = (
        jnp.dot(g / d, wcls_ref[...], preferred_element_type=f32)
        + bcls_ref[...]
    )


def kernel(x, adj, W1, b1, W2, b2, Watt, batt, Wcls, bcls):
    N, DIN = x.shape
    H = W1.shape[1]
    C = Wcls.shape[1]
    f32 = jnp.float32

    a = pl.pallas_call(
        _fc1_body,
        out_shape=jax.ShapeDtypeStruct((N, H), f32),
    )(x, W1, b1.reshape(1, H))

    out = pl.pallas_call(
        functools.partial(_main_body, n=N, h=H, c=C),
        in_specs=[
            pl.BlockSpec(memory_space=pltpu.MemorySpace.VMEM),
            pl.BlockSpec(memory_space=pl.ANY),
            pl.BlockSpec(memory_space=pltpu.MemorySpace.VMEM),
            pl.BlockSpec(memory_space=pltpu.MemorySpace.VMEM),
            pl.BlockSpec(memory_space=pltpu.MemorySpace.VMEM),
            pl.BlockSpec(memory_space=pltpu.MemorySpace.VMEM),
            pl.BlockSpec(memory_space=pltpu.MemorySpace.VMEM),
            pl.BlockSpec(memory_space=pltpu.MemorySpace.VMEM),
        ],
        out_shape=jax.ShapeDtypeStruct((1, C), f32),
        scratch_shapes=[
            pltpu.VMEM((_R, N), f32),
            pltpu.VMEM((_R, N), f32),
            pltpu.VMEM((_CACHE_CHUNKS * _R, N), f32),
            pltpu.VMEM((N, H), f32),
            pltpu.VMEM((N, H), f32),
            pltpu.SemaphoreType.DMA,
            pltpu.SemaphoreType.DMA,
            pltpu.SemaphoreType.DMA,
        ],
        compiler_params=pltpu.CompilerParams(
            vmem_limit_bytes=64 * 1024 * 1024,
        ),
    )(a, adj, W2, b2.reshape(1, H), Watt, batt.reshape(1, 1), Wcls,
      bcls.reshape(1, C))

    return out.reshape(C)
